# baseline (speedup 1.0000x reference)
import functools

import jax
import jax.numpy as jnp
from jax import lax
from jax.experimental import pallas as pl
from jax.experimental.pallas import tpu as pltpu
from jax.experimental.pallas import tpu_sc as plsc

NUM_RADIAL = 128
NUM_SPH = 8
EMB = 32
OUT_COLS = NUM_SPH * EMB

_NC = 2
_NS = 16
_NW = _NC * _NS
_CHUNK = 96
_NBUF = 8
_K = 4


def _sc_gather(table, idx, t_pad, nchunks):
    mesh = plsc.VectorSubcoreMesh(core_axis_name="c", subcore_axis_name="s")

    @functools.partial(
        pl.kernel,
        mesh=mesh,
        out_type=jax.ShapeDtypeStruct((t_pad, NUM_RADIAL), jnp.float32),
        scratch_types=[
            pltpu.VMEM((nchunks, _CHUNK), jnp.int32),
            pltpu.VMEM((_NBUF, _CHUNK, NUM_RADIAL), jnp.float32),
        ] + [pltpu.SemaphoreType.DMA] * (2 * _NBUF),
    )
    def k(table_hbm, idx_hbm, out_hbm, idx_v, rows_v, *sems):
        gsem, wsem = sems[:_NBUF], sems[_NBUF:]
        wid = lax.axis_index("s") * _NC + lax.axis_index("c")
        base = wid * nchunks
        pltpu.sync_copy(idx_hbm.at[wid], idx_v)

        def start_g(c, b):
            pltpu.async_copy(table_hbm.at[idx_v.at[c]], rows_v.at[b], gsem[b])

        def drain_g(b):
            pltpu.make_async_copy(
                table_hbm.at[pl.ds(0, _CHUNK)], rows_v.at[b], gsem[b]).wait()

        def start_w(c, b):
            pltpu.async_copy(
                rows_v.at[b],
                out_hbm.at[pl.ds((base + c) * _CHUNK, _CHUNK)], wsem[b])

        def drain_w(b):
            pltpu.make_async_copy(
                rows_v.at[b],
                out_hbm.at[pl.ds(base * _CHUNK, _CHUNK)], wsem[b]).wait()

        def step(c, b, warmed_up, more_ahead):
            bb = (b + _K) % _NBUF
            if warmed_up:
                drain_w(bb)
            if more_ahead:
                start_g(c + _K, bb)
            drain_g(b)
            start_w(c, b)

        for b in range(_K):
            start_g(b, b)

        for b in range(_NBUF):
            step(b, b, b >= _NBUF - _K, True)

        def body(j, carry):
            c0 = _NBUF * j
            for b in range(_NBUF):
                step(c0 + b, b, True, True)
            return carry

        lax.fori_loop(1, nchunks // _NBUF - 1, body, 0, unroll=False)

        c0 = nchunks - _NBUF
        for b in range(_NBUF):
            step(c0 + b, b, True, b < _NBUF - _K)
        for b in range(_NBUF - _K, _NBUF):
            drain_w(b)

    return k(table, idx)


def _tc_contract(g, sph, w, b_mat, p_mat, t_pad, tile):
    def body(g_ref, s_ref, w_ref, b_ref, p_ref, o_ref):
        h = jnp.dot(g_ref[...], w_ref[...], preferred_element_type=jnp.float32)
        srep = jnp.dot(s_ref[...], b_ref[...], preferred_element_type=jnp.float32)
        o_ref[...] = jnp.dot(h * srep, p_ref[...],
                             preferred_element_type=jnp.float32)

    return pl.pallas_call(
        body,
        grid=(t_pad // tile,),
        in_specs=[
            pl.BlockSpec((tile, NUM_RADIAL), lambda i: (i, 0)),
            pl.BlockSpec((tile, NUM_SPH), lambda i: (i, 0)),
            pl.BlockSpec((NUM_RADIAL, OUT_COLS), lambda i: (0, 0)),
            pl.BlockSpec((NUM_SPH, OUT_COLS), lambda i: (0, 0)),
            pl.BlockSpec((OUT_COLS, EMB), lambda i: (0, 0)),
        ],
        out_specs=pl.BlockSpec((tile, EMB), lambda i: (i, 0)),
        out_shape=jax.ShapeDtypeStruct((t_pad, EMB), jnp.float32),
    )(g, sph, w, b_mat, p_mat)


def kernel(rbf, sph, idx_sph, weight):
    t = idx_sph.shape[0]
    tile = 1024
    per_w = -(-t // (_NW * _CHUNK)) * _CHUNK
    nchunks = per_w // _CHUNK
    t_pad = _NW * per_w
    while nchunks % _NBUF or (_NW * nchunks * _CHUNK) % tile:
        nchunks += 1
    t_pad = _NW * nchunks * _CHUNK

    idx_pad = jnp.zeros((t_pad,), jnp.int32).at[:t].set(idx_sph)
    sph_pad = jnp.zeros((t_pad, NUM_SPH), sph.dtype).at[:t].set(sph)

    g = _sc_gather(rbf, idx_pad.reshape(_NW, nchunks, _CHUNK), t_pad, nchunks)

    w = weight.reshape(NUM_RADIAL, OUT_COLS)
    b_mat = jnp.tile(jnp.eye(NUM_SPH, dtype=jnp.float32), (1, EMB))
    p_mat = jnp.repeat(jnp.eye(EMB, dtype=jnp.float32), NUM_SPH, axis=0)

    out = _tc_contract(g, sph_pad, w, b_mat, p_mat, t_pad, tile)
    return out[:t]


# speedup vs baseline: 1.8702x; 1.8702x over previous
import functools

import jax
import jax.numpy as jnp
from jax import lax
from jax.experimental import pallas as pl
from jax.experimental.pallas import tpu as pltpu
from jax.experimental.pallas import tpu_sc as plsc

NUM_RADIAL = 128
NUM_SPH = 8
EMB = 32
OUT_COLS = NUM_SPH * EMB

_NC = 2
_NS = 16
_NW = _NC * _NS
_CHUNK = 128


def _sc_gather(table, idx, t_pad, nchunks):
    mesh = plsc.VectorSubcoreMesh(core_axis_name="c", subcore_axis_name="s")

    @functools.partial(
        pl.kernel,
        mesh=mesh,
        out_type=jax.ShapeDtypeStruct((t_pad, NUM_RADIAL), jnp.float32),
        scratch_types=[
            pltpu.VMEM((_CHUNK,), jnp.int32),
            pltpu.VMEM((_CHUNK,), jnp.int32),
            pltpu.VMEM((_CHUNK, NUM_RADIAL), jnp.float32),
            pltpu.VMEM((_CHUNK, NUM_RADIAL), jnp.float32),
            pltpu.SemaphoreType.DMA,
            pltpu.SemaphoreType.DMA,
            pltpu.SemaphoreType.DMA,
            pltpu.SemaphoreType.DMA,
        ],
    )
    def k(table_hbm, idx_hbm, out_hbm, idx0, idx1, rows0, rows1,
          g0, g1, w0, w1):
        wid = lax.axis_index("s") * _NC + lax.axis_index("c")
        base = wid * nchunks

        def off(c):
            return (base + c) * _CHUNK

        def do_chunk(c, idxb, rowsb, gsem, wsem, drain_first):
            pltpu.sync_copy(idx_hbm.at[pl.ds(off(c), _CHUNK)], idxb)
            if drain_first:
                pltpu.make_async_copy(
                    rowsb, out_hbm.at[pl.ds(off(c), _CHUNK)], wsem).wait()
            pltpu.async_copy(table_hbm.at[idxb], rowsb, gsem).wait()
            pltpu.async_copy(rowsb, out_hbm.at[pl.ds(off(c), _CHUNK)], wsem)

        do_chunk(0, idx0, rows0, g0, w0, False)
        do_chunk(1, idx1, rows1, g1, w1, False)

        def body(j, carry):
            do_chunk(2 * j, idx0, rows0, g0, w0, True)
            do_chunk(2 * j + 1, idx1, rows1, g1, w1, True)
            return carry

        lax.fori_loop(1, nchunks // 2, body, 0, unroll=False)
        pltpu.make_async_copy(
            rows0, out_hbm.at[pl.ds(off(nchunks - 2), _CHUNK)], w0).wait()
        pltpu.make_async_copy(
            rows1, out_hbm.at[pl.ds(off(nchunks - 1), _CHUNK)], w1).wait()

    return k(table, idx)


def _tc_contract(g, sph, w, b_mat, p_mat, t, tile):
    def body(g_ref, s_ref, w_ref, b_ref, p_ref, o_ref):
        h = jnp.dot(g_ref[...], w_ref[...], preferred_element_type=jnp.float32)
        srep = jnp.dot(s_ref[...], b_ref[...], preferred_element_type=jnp.float32)
        o_ref[...] = jnp.dot(h * srep, p_ref[...],
                             preferred_element_type=jnp.float32)

    return pl.pallas_call(
        body,
        grid=(t // tile,),
        in_specs=[
            pl.BlockSpec((tile, NUM_RADIAL), lambda i: (i, 0)),
            pl.BlockSpec((tile, NUM_SPH), lambda i: (i, 0)),
            pl.BlockSpec((NUM_RADIAL, OUT_COLS), lambda i: (0, 0)),
            pl.BlockSpec((NUM_SPH, OUT_COLS), lambda i: (0, 0)),
            pl.BlockSpec((OUT_COLS, EMB), lambda i: (0, 0)),
        ],
        out_specs=pl.BlockSpec((tile, EMB), lambda i: (i, 0)),
        out_shape=jax.ShapeDtypeStruct((t, EMB), jnp.float32),
    )(g, sph, w, b_mat, p_mat)


def kernel(rbf, sph, idx_sph, weight):
    t = idx_sph.shape[0]
    tile = 1000
    nchunks = -(-t // (_NW * _CHUNK))
    nchunks += nchunks % 2
    t_pad = _NW * nchunks * _CHUNK

    idx_pad = jnp.zeros((t_pad,), jnp.int32).at[:t].set(idx_sph)

    g = _sc_gather(rbf, idx_pad, t_pad, nchunks)

    w = weight.reshape(NUM_RADIAL, OUT_COLS)
    b_mat = jnp.tile(jnp.eye(NUM_SPH, dtype=jnp.float32), (1, EMB))
    p_mat = jnp.repeat(jnp.eye(EMB, dtype=jnp.float32), NUM_SPH, axis=0)

    return _tc_contract(g, sph, w, b_mat, p_mat, t, tile)


# speedup vs baseline: 1.8924x; 1.0119x over previous
import functools

import jax
import jax.numpy as jnp
from jax import lax
from jax.experimental import pallas as pl
from jax.experimental.pallas import tpu as pltpu
from jax.experimental.pallas import tpu_sc as plsc

NUM_RADIAL = 128
NUM_SPH = 8
EMB = 32
OUT_COLS = NUM_SPH * EMB

_NC = 2
_NS = 16
_NW = _NC * _NS
_CHUNK = 128


def _sc_gather(table, idx, t_pad, nc0, nc1):
    mesh = plsc.VectorSubcoreMesh(core_axis_name="c", subcore_axis_name="s")

    @functools.partial(
        pl.kernel,
        mesh=mesh,
        out_type=jax.ShapeDtypeStruct((t_pad, NUM_RADIAL), jnp.float32),
        scratch_types=[
            pltpu.VMEM((_CHUNK,), jnp.int32),
            pltpu.VMEM((_CHUNK,), jnp.int32),
            pltpu.VMEM((_CHUNK, NUM_RADIAL), jnp.float32),
            pltpu.VMEM((_CHUNK, NUM_RADIAL), jnp.float32),
            pltpu.SemaphoreType.DMA,
            pltpu.SemaphoreType.DMA,
            pltpu.SemaphoreType.DMA,
            pltpu.SemaphoreType.DMA,
        ],
    )
    def k(table_hbm, idx_hbm, out_hbm, idx0, idx1, rows0, rows1,
          g0, g1, w0, w1):
        c_ax = lax.axis_index("c")
        s_ax = lax.axis_index("s")
        my_n = jnp.where(c_ax == 0, nc0, nc1)
        base = jnp.where(c_ax == 0, s_ax * nc0, _NS * nc0 + s_ax * nc1)

        def off(c):
            return (base + c) * _CHUNK

        def do_chunk(c, idxb, rowsb, gsem, wsem, drain_first):
            pltpu.sync_copy(idx_hbm.at[pl.ds(off(c), _CHUNK)], idxb)
            if drain_first:
                pltpu.make_async_copy(
                    rowsb, out_hbm.at[pl.ds(off(c), _CHUNK)], wsem).wait()
            pltpu.async_copy(table_hbm.at[idxb], rowsb, gsem).wait()
            pltpu.async_copy(rowsb, out_hbm.at[pl.ds(off(c), _CHUNK)], wsem)

        do_chunk(0, idx0, rows0, g0, w0, False)
        do_chunk(1, idx1, rows1, g1, w1, False)

        def body(j, carry):
            do_chunk(2 * j, idx0, rows0, g0, w0, True)
            do_chunk(2 * j + 1, idx1, rows1, g1, w1, True)
            return carry

        lax.fori_loop(1, my_n // 2, body, 0, unroll=False)
        pltpu.make_async_copy(
            rows0, out_hbm.at[pl.ds(off(my_n - 2), _CHUNK)], w0).wait()
        pltpu.make_async_copy(
            rows1, out_hbm.at[pl.ds(off(my_n - 1), _CHUNK)], w1).wait()

    return k(table, idx)


def _tc_contract(g, sph, w, b_mat, p_mat, t, tile):
    def body(g_ref, s_ref, w_ref, b_ref, p_ref, o_ref):
        h = jnp.dot(g_ref[...], w_ref[...], preferred_element_type=jnp.float32)
        srep = jnp.dot(s_ref[...], b_ref[...], preferred_element_type=jnp.float32)
        o_ref[...] = jnp.dot(h * srep, p_ref[...],
                             preferred_element_type=jnp.float32)

    return pl.pallas_call(
        body,
        grid=(t // tile,),
        in_specs=[
            pl.BlockSpec((tile, NUM_RADIAL), lambda i: (i, 0)),
            pl.BlockSpec((tile, NUM_SPH), lambda i: (i, 0)),
            pl.BlockSpec((NUM_RADIAL, OUT_COLS), lambda i: (0, 0)),
            pl.BlockSpec((NUM_SPH, OUT_COLS), lambda i: (0, 0)),
            pl.BlockSpec((OUT_COLS, EMB), lambda i: (0, 0)),
        ],
        out_specs=pl.BlockSpec((tile, EMB), lambda i: (i, 0)),
        out_shape=jax.ShapeDtypeStruct((t, EMB), jnp.float32),
    )(g, sph, w, b_mat, p_mat)


def kernel(rbf, sph, idx_sph, weight):
    t = idx_sph.shape[0]
    tile = 1000
    nchunks = -(-t // (_NW * _CHUNK))
    nchunks += nchunks % 2
    t_pad = _NW * nchunks * _CHUNK
    nc0 = max(4, (2 * nchunks * 13 // 20) // 2 * 2)
    nc1 = 2 * nchunks - nc0

    idx_pad = jnp.zeros((t_pad,), jnp.int32).at[:t].set(idx_sph)

    g = _sc_gather(rbf, idx_pad, t_pad, nc0, nc1)

    w = weight.reshape(NUM_RADIAL, OUT_COLS)
    b_mat = jnp.tile(jnp.eye(NUM_SPH, dtype=jnp.float32), (1, EMB))
    p_mat = jnp.repeat(jnp.eye(EMB, dtype=jnp.float32), NUM_SPH, axis=0)

    return _tc_contract(g, sph, w, b_mat, p_mat, t, tile)


# speedup vs baseline: 1.8934x; 1.0005x over previous
import functools

import jax
import jax.numpy as jnp
from jax import lax
from jax.experimental import pallas as pl
from jax.experimental.pallas import tpu as pltpu
from jax.experimental.pallas import tpu_sc as plsc

NUM_RADIAL = 128
NUM_SPH = 8
EMB = 32
OUT_COLS = NUM_SPH * EMB

_NC = 2
_NS = 16
_NW = _NC * _NS
_CHUNK = 128


def _sc_gather(table, idx, t_pad, nc0, nc1):
    mesh = plsc.VectorSubcoreMesh(core_axis_name="c", subcore_axis_name="s")

    @functools.partial(
        pl.kernel,
        mesh=mesh,
        out_type=jax.ShapeDtypeStruct((t_pad, NUM_RADIAL), jnp.float32),
        scratch_types=[
            pltpu.VMEM((_CHUNK,), jnp.int32),
            pltpu.VMEM((_CHUNK,), jnp.int32),
            pltpu.VMEM((_CHUNK, NUM_RADIAL), jnp.float32),
            pltpu.VMEM((_CHUNK, NUM_RADIAL), jnp.float32),
            pltpu.SemaphoreType.DMA,
            pltpu.SemaphoreType.DMA,
            pltpu.SemaphoreType.DMA,
            pltpu.SemaphoreType.DMA,
        ],
    )
    def k(table_hbm, idx_hbm, out_hbm, idx0, idx1, rows0, rows1,
          g0, g1, w0, w1):
        c_ax = lax.axis_index("c")
        s_ax = lax.axis_index("s")
        my_n = jnp.where(c_ax == 0, nc0, nc1)
        base = jnp.where(c_ax == 0, s_ax * nc0, _NS * nc0 + s_ax * nc1)

        def off(c):
            return (base + c) * _CHUNK

        def do_chunk(c, idxb, rowsb, gsem, wsem, drain_first):
            pltpu.sync_copy(idx_hbm.at[pl.ds(off(c), _CHUNK)], idxb)
            if drain_first:
                pltpu.make_async_copy(
                    rowsb, out_hbm.at[pl.ds(off(c), _CHUNK)], wsem).wait()
            pltpu.async_copy(table_hbm.at[idxb], rowsb, gsem).wait()
            pltpu.async_copy(rowsb, out_hbm.at[pl.ds(off(c), _CHUNK)], wsem)

        do_chunk(0, idx0, rows0, g0, w0, False)
        do_chunk(1, idx1, rows1, g1, w1, False)

        def body(j, carry):
            do_chunk(2 * j, idx0, rows0, g0, w0, True)
            do_chunk(2 * j + 1, idx1, rows1, g1, w1, True)
            return carry

        lax.fori_loop(1, my_n // 2, body, 0, unroll=False)
        pltpu.make_async_copy(
            rows0, out_hbm.at[pl.ds(off(my_n - 2), _CHUNK)], w0).wait()
        pltpu.make_async_copy(
            rows1, out_hbm.at[pl.ds(off(my_n - 1), _CHUNK)], w1).wait()

    return k(table, idx)


def _tc_contract(g, sph, w, b_mat, p_mat, t, tile):
    def body(g_ref, s_ref, w_ref, b_ref, p_ref, o_ref):
        gb = g_ref[...].astype(jnp.bfloat16)
        h = jnp.dot(gb, w_ref[...], preferred_element_type=jnp.float32)
        srep = jnp.dot(s_ref[...], b_ref[...], preferred_element_type=jnp.float32)
        hs = (h * srep).astype(jnp.bfloat16)
        o_ref[...] = jnp.dot(hs, p_ref[...],
                             preferred_element_type=jnp.float32)

    return pl.pallas_call(
        body,
        grid=(t // tile,),
        in_specs=[
            pl.BlockSpec((tile, NUM_RADIAL), lambda i: (i, 0)),
            pl.BlockSpec((tile, NUM_SPH), lambda i: (i, 0)),
            pl.BlockSpec((NUM_RADIAL, OUT_COLS), lambda i: (0, 0)),
            pl.BlockSpec((NUM_SPH, OUT_COLS), lambda i: (0, 0)),
            pl.BlockSpec((OUT_COLS, EMB), lambda i: (0, 0)),
        ],
        out_specs=pl.BlockSpec((tile, EMB), lambda i: (i, 0)),
        out_shape=jax.ShapeDtypeStruct((t, EMB), jnp.float32),
    )(g, sph, w, b_mat, p_mat)


def kernel(rbf, sph, idx_sph, weight):
    t = idx_sph.shape[0]
    tile = 1000
    nchunks = -(-t // (_NW * _CHUNK))
    nchunks += nchunks % 2
    t_pad = _NW * nchunks * _CHUNK
    nc0 = max(4, (2 * nchunks * 13 // 20) // 2 * 2)
    nc1 = 2 * nchunks - nc0

    idx_pad = jnp.zeros((t_pad,), jnp.int32).at[:t].set(idx_sph)

    g = _sc_gather(rbf, idx_pad, t_pad, nc0, nc1)

    w = weight.reshape(NUM_RADIAL, OUT_COLS).astype(jnp.bfloat16)
    b_mat = jnp.tile(jnp.eye(NUM_SPH, dtype=jnp.float32), (1, EMB))
    p_mat = jnp.repeat(jnp.eye(EMB, dtype=jnp.bfloat16), NUM_SPH, axis=0)

    return _tc_contract(g, sph, w, b_mat, p_mat, t, tile)
